# baseline (speedup 1.0000x reference)
import functools
import math

import jax
import jax.numpy as jnp
import numpy as np
from jax import lax
from jax.experimental import pallas as pl
from jax.experimental.pallas import tpu as pltpu
from jax.experimental.pallas import tpu_sc as plsc

SPACE_DIMS = 3
N_SEG = 16
ROW = 2048
LANES = 16
NITER = ROW // LANES
NC = 2
NS = 16
SEG_PER_CORE = N_SEG // NC

def _deg2dist(deg):
    return 2.0 * np.sin(np.radians(np.asarray(deg, dtype=np.float64)) / 2.0)

_T_MIN = np.float32(_deg2dist(10.0 / 3600.0) ** 2)
_T_MAX = np.float32(_deg2dist(1.0) ** 2)
_LOG_RANGE = np.float32(np.log(np.float64(_T_MAX) / np.float64(_T_MIN)))
_SIGMA = np.float32(np.e)
_INV_SIGMA = np.float32(1.0) / _SIGMA
_COEF = np.float32(np.float32(1.0 / np.sqrt(2.0 * np.pi)) / _SIGMA)
_LN2 = np.float32(0.693147180559945309)
_LOG_1EM30 = np.float32(np.log(1e-30))


def _logf(x):
    ix = lax.bitcast_convert_type(x, jnp.int32)
    ix = ix + (0x3F800000 - 0x3F3504F3)
    k = lax.shift_right_arithmetic(ix, 23) - 127
    mx = (ix & 0x007FFFFF) + 0x3F3504F3
    m = lax.bitcast_convert_type(mx, jnp.float32)
    f = m - 1.0
    s = f / (2.0 + f)
    z = s * s
    w = z * z
    t1 = w * (np.float32(0.40000972152) + w * np.float32(0.24279078841))
    t2 = z * (np.float32(0.66666662693) + w * np.float32(0.28498786688))
    r = t2 + t1
    hfsq = np.float32(0.5) * f * f
    return f - (hfsq - s * (hfsq + r)) + k.astype(jnp.float32) * _LN2


@functools.partial(
    pl.kernel,
    out_type=(
        jax.ShapeDtypeStruct((N_SEG,), jnp.float32),
        jax.ShapeDtypeStruct((N_SEG,), jnp.float32),
        jax.ShapeDtypeStruct((N_SEG,), jnp.float32),
    ),
    mesh=plsc.VectorSubcoreMesh(
        core_axis_name="c", subcore_axis_name="s", num_cores=NC, num_subcores=NS
    ),
    compiler_params=pltpu.CompilerParams(needs_layout_passes=False),
    scratch_types=[
        pltpu.VMEM((ROW,), jnp.float32),
        pltpu.VMEM((ROW,), jnp.float32),
        pltpu.VMEM((ROW,), jnp.float32),
        pltpu.VMEM((ROW,), jnp.float32),
        pltpu.VMEM((ROW,), jnp.float32),
        pltpu.VMEM((ROW,), jnp.float32),
        pltpu.VMEM((ROW,), jnp.float32),
        pltpu.VMEM((ROW,), jnp.float32),
        pltpu.VMEM((LANES,), jnp.float32),
        pltpu.VMEM((LANES,), jnp.float32),
        pltpu.VMEM((LANES,), jnp.float32),
        pltpu.VMEM((LANES,), jnp.float32),
        pltpu.VMEM((3, SEG_PER_CORE, LANES), jnp.float32),
        pltpu.VMEM((LANES,), jnp.float32),
        pltpu.HBM((3, N_SEG, LANES), jnp.float32),
        pltpu.SemaphoreType.DMA,
    ],
)
def _tscore(
    up_h, uo_h, mp_h, mo_h, thp_h,
    score_h, hits_h, ll_h,
    upx, upy, upz, uox, uoy, uoz, mp, mo, thp,
    stage_p, stage_hh, stage_l, gbuf, outv, stage_sh, dsem,
):
    ci = lax.axis_index("c")
    si = lax.axis_index("s")
    active = si < SEG_PER_CORE
    seg = ci * SEG_PER_CORE + si
    lane = lax.iota(jnp.int32, LANES)

    @pl.when(active)
    def _work():
        base = seg * ROW
        sl_h = pl.ds(base, ROW)
        D = ROW * N_SEG
        cps = (
            pltpu.async_copy(up_h.at[pl.ds(base, ROW)], upx, dsem),
            pltpu.async_copy(up_h.at[pl.ds(D + base, ROW)], upy, dsem),
            pltpu.async_copy(up_h.at[pl.ds(2 * D + base, ROW)], upz, dsem),
            pltpu.async_copy(uo_h.at[pl.ds(base, ROW)], uox, dsem),
            pltpu.async_copy(uo_h.at[pl.ds(D + base, ROW)], uoy, dsem),
            pltpu.async_copy(uo_h.at[pl.ds(2 * D + base, ROW)], uoz, dsem),
            pltpu.async_copy(mp_h.at[sl_h], mp, dsem),
            pltpu.async_copy(mo_h.at[sl_h], mo, dsem),
            pltpu.async_copy(thp_h, thp, dsem),
        )
        for cp in cps:
            cp.wait()

        onehot = lane == seg
        th_all = _T_MIN * jnp.exp(thp[...] * _LOG_RANGE)
        one = jnp.float32(1.0)
        zero = jnp.float32(0.0)
        th = jnp.sum(jnp.where(onehot, th_all, zero))
        rinv = jnp.sum(jnp.where(onehot, one / th_all, zero))

        def chunk(i, accp, acch, accl, accn, acct):
            sl = pl.ds(i * LANES, LANES)
            dux = upx[sl] - uox[sl]
            duy = upy[sl] - uoy[sl]
            duz = upz[sl] - uoz[sl]
            s2 = dux * dux + duy * duy + duz * duz
            close = s2 < th
            vv = s2 * rinv
            dm = mp[sl] - mo[sl]
            zz = dm * _INV_SIGMA
            pmag = _COEF * jnp.exp(np.float32(-0.5) * zz * zz)
            cf = jnp.where(close, one, zero)
            p = jnp.where(close, (one - vv) * pmag, zero)
            pos = p > zero
            selpos = jnp.where(close & pos, one, zero)
            lp = _logf(jnp.maximum(p, jnp.float32(1e-37)))
            tiny = jnp.where(close & pos & (p < jnp.float32(1e-26)), one, zero)
            return (accp + p, acch + cf, accl + selpos * lp,
                    accn + selpos, acct + tiny)

        def body1(i, carry):
            a = chunk(4 * i, *carry)
            a = chunk(4 * i + 1, *a)
            a = chunk(4 * i + 2, *a)
            return chunk(4 * i + 3, *a)

        zero16 = jnp.zeros((LANES,), jnp.float32)
        accp, acch, accl, accn, acct = lax.fori_loop(
            0, NITER // 4, body1, (zero16,) * 5
        )
        ps = jnp.sum(accp)
        hs = jnp.sum(acch)
        npos = jnp.sum(accn)
        ntiny = jnp.sum(acct)
        den = jnp.maximum(ps, jnp.float32(1e-30))
        n0 = hs - npos
        logden = jnp.max(_logf(jnp.zeros((LANES,), jnp.float32) + den))
        ls_fast = jnp.sum(accl) - npos * logden + n0 * _LOG_1EM30

        def exact_ll(_):
            def body2(i, accl2):
                sl = pl.ds(i * LANES, LANES)
                dux = upx[sl] - uox[sl]
                duy = upy[sl] - uoy[sl]
                duz = upz[sl] - uoz[sl]
                s2 = dux * dux + duy * duy + duz * duz
                close = s2 < th
                dm = mp[sl] - mo[sl]
                zz = dm * _INV_SIGMA
                pmag = _COEF * jnp.exp(np.float32(-0.5) * zz * zz)
                p = jnp.where(close, (one - s2 * rinv) * pmag, zero)
                cf = jnp.where(close, one, zero)
                t = jnp.maximum(p / den, jnp.float32(1e-30))
                return accl2 + cf * _logf(t)

            return jnp.sum(lax.fori_loop(0, NITER, body2, zero16))

        ls = lax.cond(
            (ntiny > zero) | (den < jnp.float32(1e-6)),
            exact_ll,
            lambda _: ls_fast,
            zero,
        )

        stage_p[...] = jnp.where(onehot, ps, jnp.float32(0.0))
        stage_hh[...] = jnp.where(onehot, hs, jnp.float32(0.0))
        stage_l[...] = jnp.where(onehot, ls, jnp.float32(0.0))
        pltpu.sync_copy(stage_p, stage_sh.at[0, seg])
        pltpu.sync_copy(stage_hh, stage_sh.at[1, seg])
        pltpu.sync_copy(stage_l, stage_sh.at[2, seg])

    plsc.subcore_barrier()

    @pl.when(si == 0)
    def _gather():
        half = pl.ds(ci * SEG_PER_CORE, SEG_PER_CORE)
        cpg = tuple(
            pltpu.async_copy(stage_sh.at[g, half], gbuf.at[g], dsem)
            for g in range(3)
        )
        for cp in cpg:
            cp.wait()
        for g, out_h in ((0, score_h), (1, hits_h), (2, ll_h)):
            acc = jnp.zeros((LANES,), jnp.float32)
            for i in range(SEG_PER_CORE):
                acc = acc + gbuf[g, i]
            outv[...] = acc
            pltpu.sync_copy(outv.at[half], out_h.at[half])


def kernel(u_pred, mag_pred, u_obs, mag_obs, thresh_s2_param):
    return _tscore(
        u_pred.T.reshape(-1), u_obs.T.reshape(-1),
        mag_pred, mag_obs, thresh_s2_param,
    )


# speedup vs baseline: 1.0480x; 1.0480x over previous
import functools

import jax
import jax.numpy as jnp
import numpy as np
from jax import lax
from jax.experimental import pallas as pl
from jax.experimental.pallas import tpu as pltpu
from jax.experimental.pallas import tpu_sc as plsc

N_SEG = 16
ROW = 2048
LANES = 16
NITER = ROW // LANES
NC = 2
NS = 16
SEG_PER_CORE = N_SEG // NC

def _deg2dist(deg):
    return 2.0 * np.sin(np.radians(np.asarray(deg, dtype=np.float64)) / 2.0)

_T_MIN = np.float32(_deg2dist(10.0 / 3600.0) ** 2)
_T_MAX = np.float32(_deg2dist(1.0) ** 2)
_LOG_RANGE = np.float32(np.log(np.float64(_T_MAX) / np.float64(_T_MIN)))
_SIGMA = np.float32(np.e)
_INV_SIGMA = np.float32(1.0) / _SIGMA
_COEF = np.float32(np.float32(1.0 / np.sqrt(2.0 * np.pi)) / _SIGMA)
_LN2 = np.float32(0.693147180559945309)
_LOG_1EM30 = np.float32(np.log(1e-30))


def _logf(x):
    ix = lax.bitcast_convert_type(x, jnp.int32)
    ix = ix + (0x3F800000 - 0x3F3504F3)
    k = lax.shift_right_arithmetic(ix, 23) - 127
    mx = (ix & 0x007FFFFF) + 0x3F3504F3
    m = lax.bitcast_convert_type(mx, jnp.float32)
    f = m - 1.0
    s = f / (2.0 + f)
    z = s * s
    w = z * z
    t1 = w * (np.float32(0.40000972152) + w * np.float32(0.24279078841))
    t2 = z * (np.float32(0.66666662693) + w * np.float32(0.28498786688))
    r = t2 + t1
    hfsq = np.float32(0.5) * f * f
    return f - (hfsq - s * (hfsq + r)) + k.astype(jnp.float32) * _LN2


@functools.partial(
    pl.kernel,
    out_type=(
        jax.ShapeDtypeStruct((N_SEG,), jnp.float32),
        jax.ShapeDtypeStruct((N_SEG,), jnp.float32),
        jax.ShapeDtypeStruct((N_SEG,), jnp.float32),
    ),
    mesh=plsc.VectorSubcoreMesh(
        core_axis_name="c", subcore_axis_name="s", num_cores=NC, num_subcores=NS
    ),
    compiler_params=pltpu.CompilerParams(needs_layout_passes=False),
    scratch_types=[
        pltpu.VMEM((ROW,), jnp.float32),
        pltpu.VMEM((ROW,), jnp.float32),
        pltpu.VMEM((ROW,), jnp.float32),
        pltpu.VMEM((ROW,), jnp.float32),
        pltpu.VMEM((ROW,), jnp.float32),
        pltpu.VMEM((ROW,), jnp.float32),
        pltpu.VMEM((ROW,), jnp.float32),
        pltpu.VMEM((ROW,), jnp.float32),
        pltpu.VMEM((LANES,), jnp.float32),
        pltpu.VMEM((LANES,), jnp.float32),
        pltpu.VMEM((LANES,), jnp.float32),
        pltpu.VMEM((LANES,), jnp.float32),
        pltpu.VMEM((3, SEG_PER_CORE, LANES), jnp.float32),
        pltpu.VMEM((LANES,), jnp.float32),
        pltpu.HBM((3, N_SEG, LANES), jnp.float32),
        pltpu.SemaphoreType.DMA,
    ],
)
def _tscore(
    up_h, uo_h, mp_h, mo_h, thp_h,
    score_h, hits_h, ll_h,
    upx, upy, upz, uox, uoy, uoz, mp, mo, thp,
    stage_p, stage_hh, stage_l, gbuf, outv, stage_sh, dsem,
):
    ci = lax.axis_index("c")
    si = lax.axis_index("s")
    active = si < SEG_PER_CORE
    seg = ci * SEG_PER_CORE + si
    lane = lax.iota(jnp.int32, LANES)

    @pl.when(active)
    def _work():
        base = seg * ROW
        sl_h = pl.ds(base, ROW)
        D = ROW * N_SEG
        cps = (
            pltpu.async_copy(up_h.at[pl.ds(base, ROW)], upx, dsem),
            pltpu.async_copy(up_h.at[pl.ds(D + base, ROW)], upy, dsem),
            pltpu.async_copy(up_h.at[pl.ds(2 * D + base, ROW)], upz, dsem),
            pltpu.async_copy(uo_h.at[pl.ds(base, ROW)], uox, dsem),
            pltpu.async_copy(uo_h.at[pl.ds(D + base, ROW)], uoy, dsem),
            pltpu.async_copy(uo_h.at[pl.ds(2 * D + base, ROW)], uoz, dsem),
            pltpu.async_copy(mp_h.at[sl_h], mp, dsem),
            pltpu.async_copy(mo_h.at[sl_h], mo, dsem),
            pltpu.async_copy(thp_h, thp, dsem),
        )
        for cp in cps:
            cp.wait()

        onehot = lane == seg
        th_all = _T_MIN * jnp.exp(thp[...] * _LOG_RANGE)
        one = jnp.float32(1.0)
        zero = jnp.float32(0.0)
        th = jnp.sum(jnp.where(onehot, th_all, zero))
        rinv = jnp.sum(jnp.where(onehot, one / th_all, zero))

        def chunk(i, accp, acch, accl, accn, acct):
            sl = pl.ds(i * LANES, LANES)
            dux = upx[sl] - uox[sl]
            duy = upy[sl] - uoy[sl]
            duz = upz[sl] - uoz[sl]
            s2 = dux * dux + duy * duy + duz * duz
            close = s2 < th
            vv = s2 * rinv
            dm = mp[sl] - mo[sl]
            zz = dm * _INV_SIGMA
            pmag = _COEF * jnp.exp(np.float32(-0.5) * zz * zz)
            cf = jnp.where(close, one, zero)
            p = jnp.where(close, (one - vv) * pmag, zero)
            pos = p > zero
            selpos = jnp.where(close & pos, one, zero)
            lp = _logf(jnp.maximum(p, jnp.float32(1e-37)))
            tiny = jnp.where(close & pos & (p < jnp.float32(1e-26)), one, zero)
            return (accp + p, acch + cf, accl + selpos * lp,
                    accn + selpos, acct + tiny)

        def body1(i, carry):
            a = chunk(4 * i, *carry)
            a = chunk(4 * i + 1, *a)
            a = chunk(4 * i + 2, *a)
            return chunk(4 * i + 3, *a)

        zero16 = jnp.zeros((LANES,), jnp.float32)
        accp, acch, accl, accn, acct = lax.fori_loop(
            0, NITER // 4, body1, (zero16,) * 5
        )
        ps = jnp.sum(accp)
        hs = jnp.sum(acch)
        npos = jnp.sum(accn)
        ntiny = jnp.sum(acct)
        den = jnp.maximum(ps, jnp.float32(1e-30))
        n0 = hs - npos
        logden = jnp.max(_logf(jnp.zeros((LANES,), jnp.float32) + den))
        ls_fast = jnp.sum(accl) - npos * logden + n0 * _LOG_1EM30

        def exact_ll(_):
            def body2(i, accl2):
                sl = pl.ds(i * LANES, LANES)
                dux = upx[sl] - uox[sl]
                duy = upy[sl] - uoy[sl]
                duz = upz[sl] - uoz[sl]
                s2 = dux * dux + duy * duy + duz * duz
                close = s2 < th
                dm = mp[sl] - mo[sl]
                zz = dm * _INV_SIGMA
                pmag = _COEF * jnp.exp(np.float32(-0.5) * zz * zz)
                p = jnp.where(close, (one - s2 * rinv) * pmag, zero)
                cf = jnp.where(close, one, zero)
                t = jnp.maximum(p / den, jnp.float32(1e-30))
                return accl2 + cf * _logf(t)

            return jnp.sum(lax.fori_loop(0, NITER, body2, zero16))

        ls = lax.cond(
            (ntiny > zero) | (den < jnp.float32(1e-6)),
            exact_ll,
            lambda _: ls_fast,
            zero,
        )

        stage_p[...] = jnp.where(onehot, ps, jnp.float32(0.0))
        stage_hh[...] = jnp.where(onehot, hs, jnp.float32(0.0))
        stage_l[...] = jnp.where(onehot, ls, jnp.float32(0.0))
        pltpu.sync_copy(stage_p, stage_sh.at[0, seg])
        pltpu.sync_copy(stage_hh, stage_sh.at[1, seg])
        pltpu.sync_copy(stage_l, stage_sh.at[2, seg])

    plsc.subcore_barrier()

    @pl.when(si == 0)
    def _gather():
        half = pl.ds(ci * SEG_PER_CORE, SEG_PER_CORE)
        cpg = tuple(
            pltpu.async_copy(stage_sh.at[g, half], gbuf.at[g], dsem)
            for g in range(3)
        )
        for cp in cpg:
            cp.wait()
        for g, out_h in ((0, score_h), (1, hits_h), (2, ll_h)):
            acc = jnp.zeros((LANES,), jnp.float32)
            for i in range(SEG_PER_CORE):
                acc = acc + gbuf[g, i]
            outv[...] = acc
            pltpu.sync_copy(outv.at[half], out_h.at[half])


def kernel(u_pred, mag_pred, u_obs, mag_obs, thresh_s2_param):
    return _tscore(
        u_pred.T.reshape(-1), u_obs.T.reshape(-1),
        mag_pred, mag_obs, thresh_s2_param,
    )
